# baseline (speedup 1.0000x reference)
import functools
import jax
import jax.numpy as jnp
from jax import lax
from jax.experimental import pallas as pl
from jax.experimental.pallas import tpu as pltpu
from jax.experimental.pallas import tpu_sc as plsc

T, B, D = 2048, 16, 512
T2H = T // 2
RPB = 2 * B
NW = 32
BLKS_PER_W = T2H // NW
NBUF = 4
PD = 2

PERM = [16 * (j % 2) + j // 2 for j in range(RPB)]

_mesh = plsc.VectorSubcoreMesh(
    core_axis_name="c", subcore_axis_name="s", num_cores=2, num_subcores=16
)


@functools.partial(
    pl.kernel,
    out_type=(
        jax.ShapeDtypeStruct((T2H, B, 2 * D), jnp.float32),
        jax.ShapeDtypeStruct((B,), jnp.int32),
    ),
    mesh=_mesh,
    compiler_params=pltpu.CompilerParams(use_tc_tiling_on_sc=False),
    scratch_types=[
        pltpu.VMEM((RPB + 1, D), jnp.float32),
        pltpu.VMEM((RPB + 1, D), jnp.float32),
        pltpu.VMEM((RPB + 1, D), jnp.float32),
        pltpu.VMEM((RPB + 1, D), jnp.float32),
        pltpu.VMEM((B,), jnp.int32),
        pltpu.VMEM((B,), jnp.int32),
        pltpu.SemaphoreType.DMA,
        pltpu.SemaphoreType.DMA,
        pltpu.SemaphoreType.DMA,
        pltpu.SemaphoreType.DMA,
        pltpu.SemaphoreType.DMA,
        pltpu.SemaphoreType.DMA,
        pltpu.SemaphoreType.DMA,
        pltpu.SemaphoreType.DMA,
    ],
)
def _shuffle(rows_hbm, lens_hbm, out_hbm, newlens_hbm, b0, b1, b2, b3,
             lens_v, nl_v, is0, is1, is2, is3, os0, os1, os2, os3):
    wid = lax.axis_index("s") * 2 + lax.axis_index("c")
    t2_base = wid * BLKS_PER_W
    bufs = (b0, b1, b2, b3)
    isems = (is0, is1, is2, is3)
    osems = (os0, os1, os2, os3)

    pltpu.sync_copy(lens_hbm, lens_v)
    nl = lens_v[...] >> 1
    nl_v[...] = nl

    @pl.when(wid == 0)
    def _():
        pltpu.sync_copy(nl_v, newlens_hbm)

    nlv = nl_v[...]
    nls = [nlv[b] for b in range(B)]

    zv = jnp.zeros((16,), jnp.float32)
    for n in range(NBUF):
        for c in range(D // 16):
            bufs[n][RPB, pl.ds(c * 16, 16)] = zv

    def valid_count(t2):
        k = jnp.int32(0)
        for b in range(B):
            k = k + jnp.where(t2 < nls[b], 1, 0).astype(jnp.int32)
        return k

    def in_rows(it, s, fire):
        t2 = t2_base + it
        base = t2 * RPB
        k = valid_count(t2)
        for half in (0, B):
            cp8 = pltpu.make_async_copy(
                rows_hbm.at[pl.ds(base + half, 8)],
                bufs[s].at[pl.ds(half, 8)],
                isems[s],
            )
            if fire:
                cp8.start()
            else:
                cp8.wait()

            @pl.when(k > 8)
            def _(half=half):
                cp16 = pltpu.make_async_copy(
                    rows_hbm.at[pl.ds(base + half + 8, 8)],
                    bufs[s].at[pl.ds(half + 8, 8)],
                    isems[s],
                )
                if fire:
                    cp16.start()
                else:
                    cp16.wait()

    def fire_outs(it, s):
        t2 = t2_base + it
        for j in range(RPB):
            src = jnp.where(t2 < nls[j // 2], PERM[j], RPB)
            pltpu.async_copy(
                bufs[s].at[src],
                out_hbm.at[t2, j // 2, pl.ds((j % 2) * D, D)],
                osems[s],
            )

    def drain_outs(s):
        pltpu.make_async_copy(
            rows_hbm.at[pl.ds(0, RPB)], bufs[s].at[pl.ds(0, RPB)], osems[s]
        ).wait()

    def step(i, s, drain, pre):
        if drain:
            drain_outs((s + PD) % NBUF)
        if pre:
            in_rows(i + PD, (s + PD) % NBUF, fire=True)
        in_rows(i, s, fire=False)
        fire_outs(i, s)

    in_rows(0, 0, fire=True)
    in_rows(1, 1, fire=True)
    step(0, 0, False, True)
    step(1, 1, False, True)
    step(2, 2, True, True)
    step(3, 3, True, True)

    def body(g, carry):
        i = 4 * g
        step(i + 0, 0, True, True)
        step(i + 1, 1, True, True)
        step(i + 2, 2, True, True)
        step(i + 3, 3, True, True)
        return carry

    lax.fori_loop(1, BLKS_PER_W // 4 - 1, body, 0)

    step(28, 0, True, True)
    step(29, 1, True, True)
    step(30, 2, False, False)
    step(31, 3, False, False)
    for s in range(NBUF):
        drain_outs(s)


def kernel(padded, lens):
    rows = padded.reshape(T * B, D)
    out, newlens = _shuffle(rows, lens.astype(jnp.int32))
    return out, newlens


# speedup vs baseline: 2.6030x; 2.6030x over previous
import functools
import jax
import jax.numpy as jnp
from jax import lax
from jax.experimental import pallas as pl
from jax.experimental.pallas import tpu as pltpu
from jax.experimental.pallas import tpu_sc as plsc

T, B, D = 2048, 16, 512
T2H = T // 2
RPB = 2 * B
NW = 32
BLKS_PER_W = T2H // NW
NBUF = 4
PD = 2

PERM = [16 * (j % 2) + j // 2 for j in range(RPB)]

_mesh = plsc.VectorSubcoreMesh(
    core_axis_name="c", subcore_axis_name="s", num_cores=2, num_subcores=16
)


@functools.partial(
    pl.kernel,
    out_type=(
        jax.ShapeDtypeStruct((T2H, B, 2 * D), jnp.float32),
        jax.ShapeDtypeStruct((B,), jnp.int32),
    ),
    mesh=_mesh,
    scratch_types=[
        pltpu.VMEM((RPB, D), jnp.float32),
        pltpu.VMEM((RPB, D), jnp.float32),
        pltpu.VMEM((RPB, D), jnp.float32),
        pltpu.VMEM((RPB, D), jnp.float32),
        pltpu.VMEM((1, 2 * D), jnp.float32),
        pltpu.VMEM((B,), jnp.int32),
        pltpu.VMEM((B,), jnp.int32),
        pltpu.SemaphoreType.DMA,
        pltpu.SemaphoreType.DMA,
        pltpu.SemaphoreType.DMA,
        pltpu.SemaphoreType.DMA,
        pltpu.SemaphoreType.DMA,
        pltpu.SemaphoreType.DMA,
        pltpu.SemaphoreType.DMA,
        pltpu.SemaphoreType.DMA,
    ],
)
def _shuffle(rows_hbm, lens_hbm, out_hbm, newlens_hbm, b0, b1, b2, b3,
             zpair, lens_v, nl_v, is0, is1, is2, is3, os0, os1, os2, os3):
    wid = lax.axis_index("s") * 2 + lax.axis_index("c")
    t2_base = wid * BLKS_PER_W
    bufs = (b0, b1, b2, b3)
    isems = (is0, is1, is2, is3)
    osems = (os0, os1, os2, os3)

    pltpu.sync_copy(lens_hbm, lens_v)
    nl = lens_v[...] >> 1
    nl_v[...] = nl

    @pl.when(wid == 0)
    def _():
        pltpu.sync_copy(nl_v, newlens_hbm)

    nlv = nl_v[...]
    nls = [nlv[b] for b in range(B)]

    zv = jnp.zeros((16,), jnp.float32)
    for c in range(2 * D // 16):
        zpair[0, pl.ds(c * 16, 16)] = zv

    def valid_count(t2):
        k = jnp.int32(0)
        for b in range(B):
            k = k + jnp.where(t2 < nls[b], 1, 0).astype(jnp.int32)
        return k

    def in_rows(it, s, fire):
        t2 = t2_base + it
        base = t2 * RPB
        k = valid_count(t2)
        for half in (0, B):
            cp8 = pltpu.make_async_copy(
                rows_hbm.at[pl.ds(base + half, 8)],
                bufs[s].at[pl.ds(half, 8)],
                isems[s],
            )
            if fire:
                cp8.start()
            else:
                cp8.wait()

            @pl.when(k > 8)
            def _(half=half):
                cp16 = pltpu.make_async_copy(
                    rows_hbm.at[pl.ds(base + half + 8, 8)],
                    bufs[s].at[pl.ds(half + 8, 8)],
                    isems[s],
                )
                if fire:
                    cp16.start()
                else:
                    cp16.wait()

    def fire_outs(it, s):
        t2 = t2_base + it
        for b in range(B):
            valid = t2 < nls[b]

            @pl.when(valid)
            def _(b=b):
                pltpu.async_copy(
                    bufs[s].at[b],
                    out_hbm.at[t2, b, pl.ds(0, D)],
                    osems[s],
                )
                pltpu.async_copy(
                    bufs[s].at[B + b],
                    out_hbm.at[t2, b, pl.ds(D, D)],
                    osems[s],
                )

            @pl.when(jnp.logical_not(valid))
            def _(b=b):
                pltpu.async_copy(
                    zpair.at[0],
                    out_hbm.at[t2, b, :],
                    osems[s],
                )

    def drain_outs(s):
        pltpu.make_async_copy(
            rows_hbm.at[pl.ds(0, RPB)], bufs[s].at[pl.ds(0, RPB)], osems[s]
        ).wait()

    def step(i, s, drain, pre):
        if drain:
            drain_outs((s + PD) % NBUF)
        if pre:
            in_rows(i + PD, (s + PD) % NBUF, fire=True)
        in_rows(i, s, fire=False)
        fire_outs(i, s)

    in_rows(0, 0, fire=True)
    in_rows(1, 1, fire=True)
    step(0, 0, False, True)
    step(1, 1, False, True)
    step(2, 2, True, True)
    step(3, 3, True, True)

    def body(g, carry):
        i = 4 * g
        step(i + 0, 0, True, True)
        step(i + 1, 1, True, True)
        step(i + 2, 2, True, True)
        step(i + 3, 3, True, True)
        return carry

    lax.fori_loop(1, BLKS_PER_W // 4 - 1, body, 0)

    step(28, 0, True, True)
    step(29, 1, True, True)
    step(30, 2, False, False)
    step(31, 3, False, False)
    for s in range(NBUF):
        drain_outs(s)


def kernel(padded, lens):
    rows = padded.reshape(T * B, D)
    out, newlens = _shuffle(rows, lens.astype(jnp.int32))
    return out, newlens
